# Initial kernel scaffold; baseline (speedup 1.0000x reference)
import jax
import jax.numpy as jnp
from jax.experimental import pallas as pl


def kernel(c_feat, g_feat, edge_index, gs_factor, cs_factor, W_mean, b_mean, W_disp, b_disp, W_pi, b_pi):
    raise NotImplementedError("write your pallas kernel here")


# speedup vs baseline: 11.6707x; 11.6707x over previous
import functools

import jax
import jax.numpy as jnp
from jax import lax
from jax.experimental import pallas as pl
from jax.experimental.pallas import tpu as pltpu
from jax.experimental.pallas import tpu_sc as plsc

N_NODES = 10000
N_EDGES = 320000
D = 128
L = 16
NC, NS = 2, 16
NW = NC * NS
EPW = N_EDGES // NW
B = 80
NCHUNK = EPW // B
NG = B // L


def _sc_body(c_hbm, g_hbm, src_hbm, dst_hbm, gs_hbm, cs_hbm, w_hbm,
             mu_out, xd_out, pi_out,
             srcv, dstv, c_rows, g_rows, gs_tab, cs_tab, wbuf,
             mu_st, xd_st, pi_st, sem_c, sem_g):
    wid = lax.axis_index("s") * NC + lax.axis_index("c")

    pltpu.sync_copy(gs_hbm, gs_tab)
    pltpu.sync_copy(cs_hbm, cs_tab)
    pltpu.sync_copy(w_hbm, wbuf)
    bvec = wbuf[3, pl.ds(0, L)]
    b_mean = bvec[0]
    b_disp = bvec[1]
    b_pi = bvec[2]
    lanes = jnp.arange(L, dtype=jnp.int32)
    w0c = [wbuf[0, pl.ds(j0 * L, L)] for j0 in range(D // L)]
    w1c = [wbuf[1, pl.ds(j0 * L, L)] for j0 in range(D // L)]
    w2c = [wbuf[2, pl.ds(j0 * L, L)] for j0 in range(D // L)]

    def chunk_body(ci, carry):
        base = wid * EPW + ci * B
        pltpu.sync_copy(src_hbm.at[pl.ds(base, B)], srcv)
        pltpu.sync_copy(dst_hbm.at[pl.ds(base, B)], dstv)
        cp_c = pltpu.async_copy(c_hbm.at[srcv], c_rows, sem_c)
        cp_g = pltpu.async_copy(g_hbm.at[dstv], g_rows, sem_g)
        cp_c.wait()
        cp_g.wait()

        def group_body(t, carry2):
            z = jnp.zeros((L,), jnp.float32)
            dm = z
            dd = z
            dp = z
            for el in range(L):
                e = t * L + el
                a0 = a1 = a2 = a3 = a4 = a5 = z
                for j0 in range(D // L):
                    jsl = pl.ds(j0 * L, L)
                    cv = c_rows[e, jsl]
                    gv = g_rows[e, jsl]
                    h = cv * gv
                    if j0 % 2 == 0:
                        a0 = a0 + h * w0c[j0]
                        a1 = a1 + h * w1c[j0]
                        a2 = a2 + h * w2c[j0]
                    else:
                        a3 = a3 + h * w0c[j0]
                        a4 = a4 + h * w1c[j0]
                        a5 = a5 + h * w2c[j0]
                msk = lanes == el
                dm = jnp.where(msk, jnp.sum(a0 + a3), dm)
                dd = jnp.where(msk, jnp.sum(a1 + a4), dd)
                dp = jnp.where(msk, jnp.sum(a2 + a5), dp)

            sl = pl.ds(t * L, L)
            gsv = plsc.load_gather(gs_tab, [dstv[sl]])
            csv = plsc.load_gather(cs_tab, [srcv[sl]])
            mu_ = gsv * (1.0 / (1.0 + jnp.exp(-(dm + b_mean))))
            mu = csv * jnp.clip(jnp.exp(mu_) - 1.0, 1e-5, 1e6)
            xd = gsv * (dd + b_disp)
            piv = 1.0 / (1.0 + jnp.exp(-(dp + b_pi)))
            mu_st[sl] = mu
            xd_st[sl] = xd
            pi_st[sl] = piv
            return carry2

        lax.fori_loop(0, NG, group_body, 0)
        pltpu.sync_copy(mu_st, mu_out.at[pl.ds(base, B)])
        pltpu.sync_copy(xd_st, xd_out.at[pl.ds(base, B)])
        pltpu.sync_copy(pi_st, pi_out.at[pl.ds(base, B)])
        return carry

    lax.fori_loop(0, NCHUNK, chunk_body, 0)


@jax.jit
def _sc_call(c_feat, g_feat, src, dst, gs, cs, wb):
    f32 = jnp.float32
    mesh = plsc.VectorSubcoreMesh(core_axis_name="c", subcore_axis_name="s")
    return pl.kernel(
        _sc_body,
        out_type=[jax.ShapeDtypeStruct((N_EDGES,), f32)] * 3,
        mesh=mesh,
        compiler_params=pltpu.CompilerParams(needs_layout_passes=False),
        scratch_types=[
            pltpu.VMEM((B,), jnp.int32),
            pltpu.VMEM((B,), jnp.int32),
            pltpu.VMEM((B, D), f32),
            pltpu.VMEM((B, D), f32),
            pltpu.VMEM((N_NODES,), f32),
            pltpu.VMEM((N_NODES,), f32),
            pltpu.VMEM((4, D), f32),
            pltpu.VMEM((B,), f32),
            pltpu.VMEM((B,), f32),
            pltpu.VMEM((B,), f32),
            pltpu.SemaphoreType.DMA,
            pltpu.SemaphoreType.DMA,
        ],
    )(c_feat, g_feat, src, dst, gs, cs, wb)


def _softplus_body(x_ref, o_ref):
    x = x_ref[...]
    sp = jnp.maximum(x, 0.0) + jnp.log1p(jnp.exp(-jnp.abs(x)))
    o_ref[...] = jnp.clip(sp, 1e-4, 1e4)


@jax.jit
def _disp_act(xd):
    x2 = xd.reshape(N_EDGES // D, D)
    out = pl.pallas_call(
        _softplus_body,
        out_shape=jax.ShapeDtypeStruct(x2.shape, jnp.float32),
    )(x2)
    return out.reshape(N_EDGES, 1)


def kernel(c_feat, g_feat, edge_index, gs_factor, cs_factor,
           W_mean, b_mean, W_disp, b_disp, W_pi, b_pi):
    src = edge_index[0].astype(jnp.int32)
    dst = edge_index[1].astype(jnp.int32)
    wb = jnp.zeros((4, D), jnp.float32)
    wb = wb.at[0].set(W_mean[0]).at[1].set(W_disp[0]).at[2].set(W_pi[0])
    wb = wb.at[3, 0].set(b_mean[0]).at[3, 1].set(b_disp[0]).at[3, 2].set(b_pi[0])
    mu, xd, piv = _sc_call(c_feat, g_feat, src, dst,
                           gs_factor[:, 0], cs_factor[:, 0], wb)
    disp = _disp_act(xd)
    return mu.reshape(N_EDGES, 1), disp, piv.reshape(N_EDGES, 1)


# speedup vs baseline: 24.2567x; 2.0784x over previous
import functools

import jax
import jax.numpy as jnp
from jax import lax
from jax.experimental import pallas as pl
from jax.experimental.pallas import tpu as pltpu
from jax.experimental.pallas import tpu_sc as plsc

N_NODES = 10000
N_EDGES = 320000
D = 128
L = 16
NC, NS = 2, 16
NW = NC * NS
EPW = N_EDGES // NW
B = 80
NCHUNK = EPW // B
NG = B // L


def _sc_body(c_hbm, g_hbm, src_hbm, dst_hbm, gs_hbm, cs_hbm, w_hbm,
             mu_out, xd_out, pi_out,
             sidx, didx, c_r0, g_r0, c_r1, g_r1, gs_tab, cs_tab, wbuf,
             mu_st, xd_st, pi_st,
             sem_c0, sem_g0, sem_c1, sem_g1):
    wid = lax.axis_index("s") * NC + lax.axis_index("c")
    c_rows = (c_r0, c_r1)
    g_rows = (g_r0, g_r1)
    sems_c = (sem_c0, sem_c1)
    sems_g = (sem_g0, sem_g1)

    pltpu.sync_copy(gs_hbm, gs_tab)
    pltpu.sync_copy(cs_hbm, cs_tab)
    pltpu.sync_copy(w_hbm, wbuf)
    pltpu.sync_copy(src_hbm.at[wid], sidx)
    pltpu.sync_copy(dst_hbm.at[wid], didx)
    bvec = wbuf[3, pl.ds(0, L)]
    b_mean = bvec[0]
    b_disp = bvec[1]
    b_pi = bvec[2]
    lanes = jnp.arange(L, dtype=jnp.int32)
    w0c = [wbuf[0, pl.ds(j0 * L, L)] for j0 in range(D // L)]
    w1c = [wbuf[1, pl.ds(j0 * L, L)] for j0 in range(D // L)]
    w2c = [wbuf[2, pl.ds(j0 * L, L)] for j0 in range(D // L)]

    def start(ci, b):
        pltpu.async_copy(c_hbm.at[sidx.at[ci]], c_rows[b], sems_c[b])
        pltpu.async_copy(g_hbm.at[didx.at[ci]], g_rows[b], sems_g[b])

    def drain(b):
        dummy = c_hbm.at[pl.ds(0, B)]
        pltpu.make_async_copy(dummy, c_rows[b], sems_c[b]).wait()
        pltpu.make_async_copy(dummy, g_rows[b], sems_g[b]).wait()

    def compute(ci, b):

        def group_body(t, carry2):
            z = jnp.zeros((L,), jnp.float32)
            dm = z
            dd = z
            dp = z
            for el in range(L):
                e = t * L + el
                a0 = a1 = a2 = a3 = a4 = a5 = z
                for j0 in range(D // L):
                    jsl = pl.ds(j0 * L, L)
                    cv = c_rows[b][e, jsl]
                    gv = g_rows[b][e, jsl]
                    h = cv * gv
                    if j0 % 2 == 0:
                        a0 = a0 + h * w0c[j0]
                        a1 = a1 + h * w1c[j0]
                        a2 = a2 + h * w2c[j0]
                    else:
                        a3 = a3 + h * w0c[j0]
                        a4 = a4 + h * w1c[j0]
                        a5 = a5 + h * w2c[j0]
                msk = lanes == el
                dm = jnp.where(msk, jnp.sum(a0 + a3), dm)
                dd = jnp.where(msk, jnp.sum(a1 + a4), dd)
                dp = jnp.where(msk, jnp.sum(a2 + a5), dp)

            sl = pl.ds(t * L, L)
            gsv = plsc.load_gather(gs_tab, [didx[ci, sl]])
            csv = plsc.load_gather(cs_tab, [sidx[ci, sl]])
            mu_ = gsv * (1.0 / (1.0 + jnp.exp(-(dm + b_mean))))
            mu = csv * jnp.clip(jnp.exp(mu_) - 1.0, 1e-5, 1e6)
            xd = gsv * (dd + b_disp)
            piv = 1.0 / (1.0 + jnp.exp(-(dp + b_pi)))
            osl = pl.ds(ci * B + t * L, L)
            mu_st[osl] = mu
            xd_st[osl] = xd
            pi_st[osl] = piv
            return carry2

        lax.fori_loop(0, NG, group_body, 0)

    start(0, 0)

    def outer(k, carry):
        ci0 = 2 * k
        start(ci0 + 1, 1)
        drain(0)
        compute(ci0, 0)
        start(ci0 + 2, 0)
        drain(1)
        compute(ci0 + 1, 1)
        return carry

    lax.fori_loop(0, (NCHUNK - 1) // 2, outer, 0)
    drain(0)
    compute(NCHUNK - 1, 0)

    obase = pl.ds(wid * EPW, EPW)
    pltpu.sync_copy(mu_st, mu_out.at[obase])
    pltpu.sync_copy(xd_st, xd_out.at[obase])
    pltpu.sync_copy(pi_st, pi_out.at[obase])


@jax.jit
def _sc_call(c_feat, g_feat, src, dst, gs, cs, wb):
    f32 = jnp.float32
    mesh = plsc.VectorSubcoreMesh(core_axis_name="c", subcore_axis_name="s")
    src3 = src.reshape(NW, NCHUNK, B)
    dst3 = dst.reshape(NW, NCHUNK, B)
    return pl.kernel(
        _sc_body,
        out_type=[jax.ShapeDtypeStruct((N_EDGES,), f32)] * 3,
        mesh=mesh,
        compiler_params=pltpu.CompilerParams(needs_layout_passes=False),
        scratch_types=[
            pltpu.VMEM((NCHUNK, B), jnp.int32),
            pltpu.VMEM((NCHUNK, B), jnp.int32),
            pltpu.VMEM((B, D), f32),
            pltpu.VMEM((B, D), f32),
            pltpu.VMEM((B, D), f32),
            pltpu.VMEM((B, D), f32),
            pltpu.VMEM((N_NODES,), f32),
            pltpu.VMEM((N_NODES,), f32),
            pltpu.VMEM((4, D), f32),
            pltpu.VMEM((EPW,), f32),
            pltpu.VMEM((EPW,), f32),
            pltpu.VMEM((EPW,), f32),
            pltpu.SemaphoreType.DMA,
            pltpu.SemaphoreType.DMA,
            pltpu.SemaphoreType.DMA,
            pltpu.SemaphoreType.DMA,
        ],
    )(c_feat, g_feat, src3, dst3, gs, cs, wb)


def _softplus_body(x_ref, o_ref):
    x = x_ref[...]
    sp = jnp.maximum(x, 0.0) + jnp.log1p(jnp.exp(-jnp.abs(x)))
    o_ref[...] = jnp.clip(sp, 1e-4, 1e4)


@jax.jit
def _disp_act(xd):
    x2 = xd.reshape(N_EDGES // D, D)
    out = pl.pallas_call(
        _softplus_body,
        out_shape=jax.ShapeDtypeStruct(x2.shape, jnp.float32),
    )(x2)
    return out.reshape(N_EDGES, 1)


def kernel(c_feat, g_feat, edge_index, gs_factor, cs_factor,
           W_mean, b_mean, W_disp, b_disp, W_pi, b_pi):
    src = edge_index[0].astype(jnp.int32)
    dst = edge_index[1].astype(jnp.int32)
    wb = jnp.zeros((4, D), jnp.float32)
    wb = wb.at[0].set(W_mean[0]).at[1].set(W_disp[0]).at[2].set(W_pi[0])
    wb = wb.at[3, 0].set(b_mean[0]).at[3, 1].set(b_disp[0]).at[3, 2].set(b_pi[0])
    mu, xd, piv = _sc_call(c_feat, g_feat, src, dst,
                           gs_factor[:, 0], cs_factor[:, 0], wb)
    disp = _disp_act(xd)
    return mu.reshape(N_EDGES, 1), disp, piv.reshape(N_EDGES, 1)
